# baseline (speedup 1.0000x reference)
import functools

import jax
import jax.numpy as jnp
from jax import lax
from jax.experimental import pallas as pl
from jax.experimental.pallas import tpu as pltpu
from jax.experimental.pallas import tpu_sc as plsc

N_NODES = 10000
N_PAD = 10240
N_EDGES = 320000
D = 128
H = 64
NC = 2
NS = 16

ROWS_PER_TILE = N_PAD // NS
EC = 128
N_ECHUNK = 160
E_PAD = NS * N_ECHUNK * EC

EDGES_PER_WORKER = N_EDGES // (NC * NS)
DEG_CHUNK = 2000
N_DEG_CHUNK = EDGES_PER_WORKER // DEG_CHUNK

BN = 1024


def _mesh():
    return plsc.VectorSubcoreMesh(
        core_axis_name="c", subcore_axis_name="s", num_cores=NC, num_subcores=NS
    )


def _deg_body(src1, dst1, src2, dst2, out, cnt, idxbuf):
    c = lax.axis_index("c")
    s = lax.axis_index("s")
    wid = s * NC + c
    zeros16 = jnp.zeros((16,), jnp.float32)
    ones16 = jnp.ones((16,), jnp.float32)

    def zbody(i, carry):
        cnt[pl.ds(i * 16, 16)] = zeros16
        return carry

    lax.fori_loop(0, (4 * N_PAD) // 16, zbody, None)

    ebase = wid * EDGES_PER_WORKER
    for a, ref in enumerate((src1, dst1, src2, dst2)):
        off = a * N_PAD

        def cbody(k, carry, ref=ref, off=off):
            pltpu.sync_copy(ref.at[pl.ds(ebase + k * DEG_CHUNK, DEG_CHUNK)], idxbuf)

            def ibody(j, inner):
                idx = idxbuf[pl.ds(j * 16, 16)]
                plsc.addupdate_scatter(cnt, [idx + off], ones16)
                return inner

            lax.fori_loop(0, DEG_CHUNK // 16, ibody, None)
            return carry

        lax.fori_loop(0, N_DEG_CHUNK, cbody, None)

    for a in range(4):
        pltpu.sync_copy(cnt.at[pl.ds(a * N_PAD, N_PAD)], out.at[a, wid])


def _sc_degrees(src1, dst1, src2, dst2):
    return pl.kernel(
        _deg_body,
        out_type=jax.ShapeDtypeStruct((4, NC * NS, N_PAD), jnp.float32),
        mesh=_mesh(),
        compiler_params=pltpu.CompilerParams(needs_layout_passes=False, use_tc_tiling_on_sc=False),
        scratch_types=[
            pltpu.VMEM((4 * N_PAD,), jnp.float32),
            pltpu.VMEM((DEG_CHUNK,), jnp.int32),
        ],
    )(src1, dst1, src2, dst2)


NBUF = 4


def _agg_body(h_a, h_b, srcr, dstr, zrows, out, sh_agg, src_all, dst_all,
              rows0, rows1, rows2, rows3,
              gsem0, gsem1, gsem2, gsem3, ssem0, ssem1, ssem2, ssem3):
    c = lax.axis_index("c")
    s = lax.axis_index("s")
    r0 = s * ROWS_PER_TILE
    pltpu.sync_copy(zrows, sh_agg.at[pl.ds(r0, ROWS_PER_TILE)])
    pltpu.sync_copy(srcr.at[pl.ds(s * N_ECHUNK, N_ECHUNK)], src_all)
    pltpu.sync_copy(dstr.at[pl.ds(s * N_ECHUNK, N_ECHUNK)], dst_all)
    plsc.subcore_barrier()

    rows = (rows0, rows1, rows2, rows3)
    gsem = (gsem0, gsem1, gsem2, gsem3)
    ssem = (ssem0, ssem1, ssem2, ssem3)

    def issue_gather(k, jb):
        @pl.when(c == 0)
        def _():
            pltpu.async_copy(h_a.at[src_all.at[k]], rows[jb], gsem[jb])

        @pl.when(c == 1)
        def _():
            pltpu.async_copy(h_b.at[src_all.at[k]], rows[jb], gsem[jb])

    def wait_gather(jb):
        pltpu.make_async_copy(h_a.at[pl.ds(0, EC)], rows[jb], gsem[jb]).wait()

    def issue_scatter(k, jb):
        pltpu.async_copy(rows[jb], sh_agg.at[dst_all.at[k]], ssem[jb], add=True)

    def wait_scatter(jb):
        pltpu.make_async_copy(h_a.at[pl.ds(0, EC)], rows[jb], ssem[jb]).wait()

    for j in range(NBUF - 1):
        issue_gather(j, j)

    def gbody(g, carry):
        for j in range(NBUF):
            k = g * NBUF + j
            pb = (j + NBUF - 1) % NBUF

            @pl.when(k + NBUF - 1 < N_ECHUNK)
            def _():
                @pl.when(k >= 1)
                def _():
                    wait_scatter(pb)

                issue_gather(k + NBUF - 1, pb)

            wait_gather(j)
            issue_scatter(k, j)
        return carry

    lax.fori_loop(0, N_ECHUNK // NBUF, gbody, None)
    for j in range(NBUF):
        wait_scatter(j)
    plsc.subcore_barrier()
    pltpu.sync_copy(sh_agg.at[pl.ds(r0, ROWS_PER_TILE)], out.at[c, pl.ds(r0, ROWS_PER_TILE)])


def _sc_agg(h_a, h_b, srcr, dstr, zrows):
    return pl.kernel(
        _agg_body,
        out_type=jax.ShapeDtypeStruct((NC, N_PAD, H), jnp.float32),
        mesh=_mesh(),
        compiler_params=pltpu.CompilerParams(needs_layout_passes=False, use_tc_tiling_on_sc=False),
        scratch_types=[
            pltpu.VMEM_SHARED((N_PAD, H), jnp.float32),
            pltpu.VMEM((N_ECHUNK, EC), jnp.int32),
            pltpu.VMEM((N_ECHUNK, EC), jnp.int32),
        ] + [pltpu.VMEM((EC, H), jnp.float32)] * NBUF
          + [pltpu.SemaphoreType.DMA] * (2 * NBUF),
    )(h_a, h_b, srcr, dstr, zrows)


def _norm_scale_body(degp_ref, x_ref, sel_ref, ha_ref, hb_ref, norms_ref):
    deg = jnp.dot(
        degp_ref[...], sel_ref[...],
        preferred_element_type=jnp.float32,
        precision=lax.Precision.HIGHEST,
    )
    norms = jnp.where(deg > 0, lax.rsqrt(jnp.maximum(deg, 1.0)), 0.0)
    norms_ref[...] = norms
    h0 = x_ref[...] * norms[:, 0:1]
    ha_ref[...] = h0[:, :H]
    hb_ref[...] = h0[:, H:]


def _tc_norm_scale(degp_t, x, sel):
    grid = (N_PAD // BN,)
    return pl.pallas_call(
        _norm_scale_body,
        grid=grid,
        in_specs=[
            pl.BlockSpec((BN, D), lambda i: (i, 0)),
            pl.BlockSpec((BN, D), lambda i: (i, 0)),
            pl.BlockSpec((D, 4), lambda i: (0, 0)),
        ],
        out_specs=[
            pl.BlockSpec((BN, H), lambda i: (i, 0)),
            pl.BlockSpec((BN, H), lambda i: (i, 0)),
            pl.BlockSpec((BN, 4), lambda i: (i, 0)),
        ],
        out_shape=[
            jax.ShapeDtypeStruct((N_PAD, H), jnp.float32),
            jax.ShapeDtypeStruct((N_PAD, H), jnp.float32),
            jax.ShapeDtypeStruct((N_PAD, 4), jnp.float32),
        ],
    )(degp_t, x, sel)


def _dense_body(agg_ref, norms_ref, w_ref, b_ref, *out_refs, pre_col, post_col,
                split_out):
    z = jnp.concatenate([agg_ref[0], agg_ref[1]], axis=1)
    z = z * norms_ref[:, pre_col:pre_col + 1]
    z = jnp.dot(
        z, w_ref[...],
        preferred_element_type=jnp.float32,
        precision=lax.Precision.HIGHEST,
    ) + b_ref[...]
    if post_col is not None:
        z = z * norms_ref[:, post_col:post_col + 1]
    z = jnp.maximum(z, 0.0)
    if split_out:
        out_refs[0][...] = z[:, :H]
        out_refs[1][...] = z[:, H:]
    else:
        out_refs[0][...] = z


def _tc_dense(agg_split, norms, w, b2d, pre_col, post_col, split_out):
    grid = (N_PAD // BN,)
    if split_out:
        out_spec = [pl.BlockSpec((BN, H), lambda i: (i, 0)),
                    pl.BlockSpec((BN, H), lambda i: (i, 0))]
        out_shape = [jax.ShapeDtypeStruct((N_PAD, H), jnp.float32),
                     jax.ShapeDtypeStruct((N_PAD, H), jnp.float32)]
    else:
        out_spec = pl.BlockSpec((BN, D), lambda i: (i, 0))
        out_shape = jax.ShapeDtypeStruct((N_PAD, D), jnp.float32)
    body = functools.partial(
        _dense_body, pre_col=pre_col, post_col=post_col, split_out=split_out
    )
    return pl.pallas_call(
        body,
        grid=grid,
        in_specs=[
            pl.BlockSpec((NC, BN, H), lambda i: (0, i, 0)),
            pl.BlockSpec((BN, 4), lambda i: (i, 0)),
            pl.BlockSpec((D, D), lambda i: (0, 0)),
            pl.BlockSpec((1, D), lambda i: (0, 0)),
        ],
        out_specs=out_spec,
        out_shape=out_shape,
    )(agg_split, norms, w, b2d)


def kernel(x, edge_index1, edge_index2, W1, b1, W2, b2):
    x = x.astype(jnp.float32)
    ei1 = edge_index1.astype(jnp.int32)
    ei2 = edge_index2.astype(jnp.int32)
    src1, dst1 = ei1[0], ei1[1]
    src2, dst2 = ei2[0], ei2[1]

    degp = _sc_degrees(src1, dst1, src2, dst2)
    degp_t = degp.transpose(2, 0, 1).reshape(N_PAD, 4 * NC * NS)
    sel = (jnp.arange(4 * NC * NS, dtype=jnp.int32)[:, None] // (NC * NS)
           == jnp.arange(4, dtype=jnp.int32)[None, :]).astype(jnp.float32)

    x_pad = jnp.pad(x, ((0, N_PAD - N_NODES), (0, 0)))
    h0_a, h0_b, norms = _tc_norm_scale(degp_t, x_pad, sel)

    pad = jnp.full((E_PAD - N_EDGES,), N_NODES, jnp.int32)
    def chunked(a):
        return jnp.concatenate([a, pad]).reshape(NS * N_ECHUNK, EC)

    zrows = jnp.zeros((ROWS_PER_TILE, H), jnp.float32)

    agg1 = _sc_agg(h0_a, h0_b, chunked(src1), chunked(dst1), zrows)
    h1_a, h1_b = _tc_dense(agg1, norms, W1, b1.reshape(1, D), 1, 2, True)
    agg2 = _sc_agg(h1_a, h1_b, chunked(src2), chunked(dst2), zrows)
    out = _tc_dense(agg2, norms, W2, b2.reshape(1, D), 3, None, False)
    return out[:N_NODES]


# speedup vs baseline: 1.0439x; 1.0439x over previous
import functools

import jax
import jax.numpy as jnp
from jax import lax
from jax.experimental import pallas as pl
from jax.experimental.pallas import tpu as pltpu
from jax.experimental.pallas import tpu_sc as plsc

N_NODES = 10000
N_PAD = 10240
N_EDGES = 320000
D = 128
H = 64
NC = 2
NS = 16

ROWS_PER_TILE = N_PAD // NS
EC = 128
N_ECHUNK = 160
E_PAD = NS * N_ECHUNK * EC

EDGES_PER_WORKER = N_EDGES // (NC * NS)
DEG_CHUNK = 2000
N_DEG_CHUNK = EDGES_PER_WORKER // DEG_CHUNK

BN = 1024


def _mesh():
    return plsc.VectorSubcoreMesh(
        core_axis_name="c", subcore_axis_name="s", num_cores=NC, num_subcores=NS
    )


def _deg_body(src1, dst1, src2, dst2, out, cnt, idxbuf):
    c = lax.axis_index("c")
    s = lax.axis_index("s")
    wid = s * NC + c
    zeros16 = jnp.zeros((16,), jnp.float32)
    ones16 = jnp.ones((16,), jnp.float32)

    def zbody(i, carry):
        cnt[pl.ds(i * 16, 16)] = zeros16
        return carry

    lax.fori_loop(0, (4 * N_PAD) // 16, zbody, None)

    ebase = wid * EDGES_PER_WORKER
    for a, ref in enumerate((src1, dst1, src2, dst2)):
        off = a * N_PAD

        def cbody(k, carry, ref=ref, off=off):
            pltpu.sync_copy(ref.at[pl.ds(ebase + k * DEG_CHUNK, DEG_CHUNK)], idxbuf)

            def ibody(j, inner):
                idx = idxbuf[pl.ds(j * 16, 16)]
                plsc.addupdate_scatter(cnt, [idx + off], ones16)
                return inner

            lax.fori_loop(0, DEG_CHUNK // 16, ibody, None)
            return carry

        lax.fori_loop(0, N_DEG_CHUNK, cbody, None)

    for a in range(4):
        pltpu.sync_copy(cnt.at[pl.ds(a * N_PAD, N_PAD)], out.at[a, wid])


def _sc_degrees(src1, dst1, src2, dst2):
    return pl.kernel(
        _deg_body,
        out_type=jax.ShapeDtypeStruct((4, NC * NS, N_PAD), jnp.float32),
        mesh=_mesh(),
        compiler_params=pltpu.CompilerParams(needs_layout_passes=False, use_tc_tiling_on_sc=False),
        scratch_types=[
            pltpu.VMEM((4 * N_PAD,), jnp.float32),
            pltpu.VMEM((DEG_CHUNK,), jnp.int32),
        ],
    )(src1, dst1, src2, dst2)


def _agg_body(h2, combr, zrows, out, sh_h, sh_agg, comb, src_i, dst_i,
              rbf0, rbf1, rf0, rf1, gsem0, gsem1, ssem0, ssem1):
    c = lax.axis_index("c")
    s = lax.axis_index("s")
    r0 = s * ROWS_PER_TILE
    pltpu.sync_copy(h2.at[c, pl.ds(r0, ROWS_PER_TILE)], sh_h.at[pl.ds(r0, ROWS_PER_TILE)])
    pltpu.sync_copy(zrows, sh_agg.at[pl.ds(r0, ROWS_PER_TILE)])
    pltpu.sync_copy(combr.at[pl.ds(s * N_ECHUNK, N_ECHUNK)], comb)
    plsc.subcore_barrier()

    rbf = (rbf0, rbf1)
    rf = (rf0, rf1)
    gsem = (gsem0, gsem1)
    ssem = (ssem0, ssem1)
    mask16 = jnp.full((16,), 0xFFFF, jnp.int32)
    sh16 = jnp.full((16,), 16, jnp.int32)

    def unpack_idx(k, jb):
        for v in range(EC // 16):
            cv = comb[k, pl.ds(v * 16, 16)]
            src_i[jb, pl.ds(v * 16, 16)] = cv & mask16
            dst_i[jb, pl.ds(v * 16, 16)] = lax.shift_right_logical(cv, sh16)

    def issue_gather(k, jb):
        unpack_idx(k, jb)
        pltpu.async_copy(sh_h.at[src_i.at[jb]], rbf[jb], gsem[jb])

    def wait_gather(jb):
        pltpu.make_async_copy(h2.at[c, pl.ds(0, EC)], rbf[jb], gsem[jb]).wait()

    def convert(jb):
        def cbody(r, carry):
            for c2 in range(H // 32):
                ab = rbf[jb][r, pl.ds(c2 * 32, 32)]
                a, b = plsc.unpack(ab, format=plsc.PackFormat.INTERLEAVED)
                rf[jb][r, pl.ds(c2 * 32, 16)] = a
                rf[jb][r, pl.ds(c2 * 32 + 16, 16)] = b
            return carry

        lax.fori_loop(0, EC, cbody, None)

    def issue_scatter(jb):
        pltpu.async_copy(rf[jb], sh_agg.at[dst_i.at[jb]], ssem[jb], add=True)

    def wait_scatter(jb):
        pltpu.make_async_copy(out.at[c, pl.ds(0, EC)], rf[jb], ssem[jb]).wait()

    issue_gather(0, 0)

    def gbody(g, carry):
        for j in (0, 1):
            k = g * 2 + j
            nb = (j + 1) % 2
            @pl.when(jnp.logical_or(g > 0, j > 0))
            def _():
                wait_scatter(nb)

            @pl.when(g * 2 + j + 1 < N_ECHUNK)
            def _():
                issue_gather(k + 1, nb)

            wait_gather(j)
            convert(j)
            issue_scatter(j)
        return carry

    lax.fori_loop(0, N_ECHUNK // 2, gbody, None)
    wait_scatter(1)
    plsc.subcore_barrier()
    pltpu.sync_copy(sh_agg.at[pl.ds(r0, ROWS_PER_TILE)], out.at[c, pl.ds(r0, ROWS_PER_TILE)])


def _sc_agg(h_split, combr, zrows):
    return pl.kernel(
        _agg_body,
        out_type=jax.ShapeDtypeStruct((NC, N_PAD, H), jnp.float32),
        mesh=_mesh(),
        compiler_params=pltpu.CompilerParams(needs_layout_passes=False, use_tc_tiling_on_sc=False),
        scratch_types=[
            pltpu.VMEM_SHARED((N_PAD, H), jnp.bfloat16),
            pltpu.VMEM_SHARED((N_PAD, H), jnp.float32),
            pltpu.VMEM((N_ECHUNK, EC), jnp.int32),
            pltpu.VMEM((2, EC), jnp.int32),
            pltpu.VMEM((2, EC), jnp.int32),
            pltpu.VMEM((EC, H), jnp.bfloat16),
            pltpu.VMEM((EC, H), jnp.bfloat16),
            pltpu.VMEM((EC, H), jnp.float32),
            pltpu.VMEM((EC, H), jnp.float32),
            pltpu.SemaphoreType.DMA,
            pltpu.SemaphoreType.DMA,
            pltpu.SemaphoreType.DMA,
            pltpu.SemaphoreType.DMA,
        ],
    )(h_split, combr, zrows)


def _norm_scale_body(degp_ref, x_ref, sel_ref, h0_ref, norms_ref):
    deg = jnp.dot(
        degp_ref[...], sel_ref[...],
        preferred_element_type=jnp.float32,
        precision=lax.Precision.HIGHEST,
    )
    norms = jnp.where(deg > 0, lax.rsqrt(jnp.maximum(deg, 1.0)), 0.0)
    norms_ref[...] = norms
    h0 = (x_ref[...] * norms[:, 0:1]).astype(jnp.bfloat16)
    h0_ref[0] = h0[:, :H]
    h0_ref[1] = h0[:, H:]


def _tc_norm_scale(degp_t, x, sel):
    grid = (N_PAD // BN,)
    return pl.pallas_call(
        _norm_scale_body,
        grid=grid,
        in_specs=[
            pl.BlockSpec((BN, D), lambda i: (i, 0)),
            pl.BlockSpec((BN, D), lambda i: (i, 0)),
            pl.BlockSpec((D, 4), lambda i: (0, 0)),
        ],
        out_specs=[
            pl.BlockSpec((NC, BN, H), lambda i: (0, i, 0)),
            pl.BlockSpec((BN, 4), lambda i: (i, 0)),
        ],
        out_shape=[
            jax.ShapeDtypeStruct((NC, N_PAD, H), jnp.bfloat16),
            jax.ShapeDtypeStruct((N_PAD, 4), jnp.float32),
        ],
    )(degp_t, x, sel)


def _dense_body(agg_ref, norms_ref, w_ref, b_ref, out_ref, *, pre_col, post_col,
                split_out):
    z = jnp.concatenate([agg_ref[0], agg_ref[1]], axis=1)
    z = z * norms_ref[:, pre_col:pre_col + 1]
    z = jnp.dot(
        z, w_ref[...],
        preferred_element_type=jnp.float32,
        precision=lax.Precision.HIGHEST,
    ) + b_ref[...]
    if post_col is not None:
        z = z * norms_ref[:, post_col:post_col + 1]
    z = jnp.maximum(z, 0.0)
    if split_out:
        zb = z.astype(jnp.bfloat16)
        out_ref[0] = zb[:, :H]
        out_ref[1] = zb[:, H:]
    else:
        out_ref[...] = z


def _tc_dense(agg_split, norms, w, b2d, pre_col, post_col, split_out):
    grid = (N_PAD // BN,)
    if split_out:
        out_spec = pl.BlockSpec((NC, BN, H), lambda i: (0, i, 0))
        out_shape = jax.ShapeDtypeStruct((NC, N_PAD, H), jnp.bfloat16)
    else:
        out_spec = pl.BlockSpec((BN, D), lambda i: (i, 0))
        out_shape = jax.ShapeDtypeStruct((N_PAD, D), jnp.float32)
    body = functools.partial(
        _dense_body, pre_col=pre_col, post_col=post_col, split_out=split_out
    )
    return pl.pallas_call(
        body,
        grid=grid,
        in_specs=[
            pl.BlockSpec((NC, BN, H), lambda i: (0, i, 0)),
            pl.BlockSpec((BN, 4), lambda i: (i, 0)),
            pl.BlockSpec((D, D), lambda i: (0, 0)),
            pl.BlockSpec((1, D), lambda i: (0, 0)),
        ],
        out_specs=out_spec,
        out_shape=out_shape,
    )(agg_split, norms, w, b2d)


def _interleave_perm():
    import numpy as np
    p = np.empty((D,), dtype=np.int32)
    for g in range(D // 32):
        base = 32 * g
        for i in range(16):
            p[base + 2 * i] = base + i
            p[base + 2 * i + 1] = base + 16 + i
    return p


def kernel(x, edge_index1, edge_index2, W1, b1, W2, b2):
    x = x.astype(jnp.float32)
    ei1 = edge_index1.astype(jnp.int32)
    ei2 = edge_index2.astype(jnp.int32)
    src1, dst1 = ei1[0], ei1[1]
    src2, dst2 = ei2[0], ei2[1]

    degp = _sc_degrees(src1, dst1, src2, dst2)
    degp_t = degp.transpose(2, 0, 1).reshape(N_PAD, 4 * NC * NS)
    sel = (jnp.arange(4 * NC * NS, dtype=jnp.int32)[:, None] // (NC * NS)
           == jnp.arange(4, dtype=jnp.int32)[None, :]).astype(jnp.float32)

    perm = jnp.asarray(_interleave_perm())
    x_pad = jnp.pad(x, ((0, N_PAD - N_NODES), (0, 0)))[:, perm]
    h0_split, norms = _tc_norm_scale(degp_t, x_pad, sel)

    pad = jnp.full((E_PAD - N_EDGES,), N_NODES, jnp.int32)
    def packed(src, dst):
        srcp = jnp.concatenate([src, pad])
        dstp = jnp.concatenate([dst, pad])
        return ((dstp << 16) | srcp).reshape(NS * N_ECHUNK, EC)

    comb1 = packed(src1, dst1)
    comb2 = packed(src2, dst2)
    zrows = jnp.zeros((ROWS_PER_TILE, H), jnp.float32)

    agg1 = _sc_agg(h0_split, comb1, zrows)
    W1p = W1[:, perm]
    b1p = b1[perm]
    h1_split = _tc_dense(agg1, norms, W1p, b1p.reshape(1, D), 1, 2, True)
    agg2 = _sc_agg(h1_split, comb2, zrows)
    out = _tc_dense(agg2, norms, W2, b2.reshape(1, D), 3, None, False)
    return out[:N_NODES]


# speedup vs baseline: 1.6028x; 1.5353x over previous
import functools

import jax
import jax.numpy as jnp
from jax import lax
from jax.experimental import pallas as pl
from jax.experimental.pallas import tpu as pltpu
from jax.experimental.pallas import tpu_sc as plsc

N_NODES = 10000
N_PAD = 10240
N_EDGES = 320000
D = 128
H = 64
NC = 2
NS = 16

ROWS_PER_TILE = N_PAD // NS
EC = 128
N_ECHUNK = 160
E_PAD = NS * N_ECHUNK * EC

EDGES_PER_WORKER = N_EDGES // (NC * NS)
DEG_CHUNK = 2000
N_DEG_CHUNK = EDGES_PER_WORKER // DEG_CHUNK

BN = 1024


def _mesh():
    return plsc.VectorSubcoreMesh(
        core_axis_name="c", subcore_axis_name="s", num_cores=NC, num_subcores=NS
    )


_UNROLL = 5


def _deg_body(src1, dst1, src2, dst2, out, cnt, idx0, idx1, isem0, isem1):
    c = lax.axis_index("c")
    s = lax.axis_index("s")
    wid = s * NC + c
    zeros16 = jnp.zeros((16,), jnp.float32)
    ones16 = jnp.ones((16,), jnp.float32)

    def zbody(i, carry):
        for u in range(8):
            cnt[pl.ds((i * 8 + u) * 16, 16)] = zeros16
        return carry

    lax.fori_loop(0, (4 * N_PAD) // (16 * 8), zbody, None)

    refs = (src1, dst1, src2, dst2)
    bufs = (idx0, idx1)
    isem = (isem0, isem1)
    ebase = wid * EDGES_PER_WORKER
    seq = [(a, k) for a in range(4) for k in range(N_DEG_CHUNK)]

    def issue_idx(t, jb):
        a, k = seq[t]
        pltpu.async_copy(
            refs[a].at[pl.ds(ebase + k * DEG_CHUNK, DEG_CHUNK)], bufs[jb], isem[jb]
        )

    def wait_idx(jb):
        pltpu.make_async_copy(src1.at[pl.ds(0, DEG_CHUNK)], bufs[jb], isem[jb]).wait()

    issue_idx(0, 0)
    for t in range(len(seq)):
        jb = t % 2
        if t + 1 < len(seq):
            issue_idx(t + 1, (t + 1) % 2)
        wait_idx(jb)
        off = seq[t][0] * N_PAD

        def ibody(j, inner, jb=jb, off=off):
            for u in range(_UNROLL):
                idx = bufs[jb][pl.ds((j * _UNROLL + u) * 16, 16)]
                plsc.addupdate_scatter(cnt, [idx + off], ones16)
            return inner

        lax.fori_loop(0, DEG_CHUNK // (16 * _UNROLL), ibody, None)

    for a in range(4):
        pltpu.sync_copy(cnt.at[pl.ds(a * N_PAD, N_PAD)], out.at[a, wid])


def _sc_degrees(src1, dst1, src2, dst2):
    return pl.kernel(
        _deg_body,
        out_type=jax.ShapeDtypeStruct((4, NC * NS, N_PAD), jnp.float32),
        mesh=_mesh(),
        compiler_params=pltpu.CompilerParams(needs_layout_passes=False, use_tc_tiling_on_sc=False),
        scratch_types=[
            pltpu.VMEM((4 * N_PAD,), jnp.float32),
            pltpu.VMEM((DEG_CHUNK,), jnp.int32),
            pltpu.VMEM((DEG_CHUNK,), jnp.int32),
            pltpu.SemaphoreType.DMA,
            pltpu.SemaphoreType.DMA,
        ],
    )(src1, dst1, src2, dst2)


def _agg_body(h2, combr, zrows, out, sh_h, sh_agg, comb, src_i, dst_i,
              rows0, rows1, gsem0, gsem1, ssem0, ssem1):
    c = lax.axis_index("c")
    s = lax.axis_index("s")
    r0 = s * ROWS_PER_TILE
    pltpu.sync_copy(h2.at[c, pl.ds(r0, ROWS_PER_TILE)], sh_h.at[pl.ds(r0, ROWS_PER_TILE)])
    pltpu.sync_copy(zrows, sh_agg.at[pl.ds(r0, ROWS_PER_TILE)])
    pltpu.sync_copy(combr.at[pl.ds(s * N_ECHUNK, N_ECHUNK)], comb)
    plsc.subcore_barrier()

    rows = (rows0, rows1)
    gsem = (gsem0, gsem1)
    ssem = (ssem0, ssem1)
    mask16 = jnp.full((16,), 0xFFFF, jnp.int32)
    sh16 = jnp.full((16,), 16, jnp.int32)

    def unpack(k, jb):
        for v in range(EC // 16):
            cv = comb[k, pl.ds(v * 16, 16)]
            src_i[jb, pl.ds(v * 16, 16)] = cv & mask16
            dst_i[jb, pl.ds(v * 16, 16)] = lax.shift_right_logical(cv, sh16)

    def issue_gather(k, jb):
        unpack(k, jb)
        pltpu.async_copy(sh_h.at[src_i.at[jb]], rows[jb], gsem[jb])

    def wait_gather(jb):
        pltpu.make_async_copy(h2.at[c, pl.ds(0, EC)], rows[jb], gsem[jb]).wait()

    def issue_scatter(jb):
        pltpu.async_copy(rows[jb], sh_agg.at[dst_i.at[jb]], ssem[jb], add=True)

    def wait_scatter(jb):
        pltpu.make_async_copy(h2.at[c, pl.ds(0, EC)], rows[jb], ssem[jb]).wait()

    issue_gather(0, 0)

    def gbody(g, carry):
        for j in (0, 1):
            k = g * 2 + j
            nb = (j + 1) % 2
            @pl.when(jnp.logical_or(g > 0, j > 0))
            def _():
                wait_scatter(nb)

            @pl.when(g * 2 + j + 1 < N_ECHUNK)
            def _():
                issue_gather(k + 1, nb)

            wait_gather(j)
            issue_scatter(j)
        return carry

    lax.fori_loop(0, N_ECHUNK // 2, gbody, None)
    wait_scatter(1)
    plsc.subcore_barrier()
    pltpu.sync_copy(sh_agg.at[pl.ds(r0, ROWS_PER_TILE)], out.at[c, pl.ds(r0, ROWS_PER_TILE)])


def _sc_agg(h_split, combr, zrows):
    return pl.kernel(
        _agg_body,
        out_type=jax.ShapeDtypeStruct((NC, N_PAD, H), jnp.float32),
        mesh=_mesh(),
        compiler_params=pltpu.CompilerParams(needs_layout_passes=False, use_tc_tiling_on_sc=False),
        scratch_types=[
            pltpu.VMEM_SHARED((N_PAD, H), jnp.float32),
            pltpu.VMEM_SHARED((N_PAD, H), jnp.float32),
            pltpu.VMEM((N_ECHUNK, EC), jnp.int32),
            pltpu.VMEM((2, EC), jnp.int32),
            pltpu.VMEM((2, EC), jnp.int32),
            pltpu.VMEM((EC, H), jnp.float32),
            pltpu.VMEM((EC, H), jnp.float32),
            pltpu.SemaphoreType.DMA,
            pltpu.SemaphoreType.DMA,
            pltpu.SemaphoreType.DMA,
            pltpu.SemaphoreType.DMA,
        ],
    )(h_split, combr, zrows)


def _norm_scale_body(degp_ref, x_ref, sel_ref, h0_ref, norms_ref):
    deg = jnp.dot(
        degp_ref[...], sel_ref[...],
        preferred_element_type=jnp.float32,
        precision=lax.Precision.HIGHEST,
    )
    norms = jnp.where(deg > 0, lax.rsqrt(jnp.maximum(deg, 1.0)), 0.0)
    norms_ref[...] = norms
    h0 = x_ref[...] * norms[:, 0:1]
    h0_ref[0] = h0[:, :H]
    h0_ref[1] = h0[:, H:]


def _tc_norm_scale(degp_t, x, sel):
    grid = (N_PAD // BN,)
    return pl.pallas_call(
        _norm_scale_body,
        grid=grid,
        in_specs=[
            pl.BlockSpec((BN, D), lambda i: (i, 0)),
            pl.BlockSpec((BN, D), lambda i: (i, 0)),
            pl.BlockSpec((D, 4), lambda i: (0, 0)),
        ],
        out_specs=[
            pl.BlockSpec((NC, BN, H), lambda i: (0, i, 0)),
            pl.BlockSpec((BN, 4), lambda i: (i, 0)),
        ],
        out_shape=[
            jax.ShapeDtypeStruct((NC, N_PAD, H), jnp.float32),
            jax.ShapeDtypeStruct((N_PAD, 4), jnp.float32),
        ],
    )(degp_t, x, sel)


def _dense_body(agg_ref, norms_ref, w_ref, b_ref, out_ref, *, pre_col, post_col,
                split_out):
    z = jnp.concatenate([agg_ref[0], agg_ref[1]], axis=1)
    z = z * norms_ref[:, pre_col:pre_col + 1]
    z = jnp.dot(
        z, w_ref[...],
        preferred_element_type=jnp.float32,
        precision=lax.Precision.HIGHEST,
    ) + b_ref[...]
    if post_col is not None:
        z = z * norms_ref[:, post_col:post_col + 1]
    z = jnp.maximum(z, 0.0)
    if split_out:
        out_ref[0] = z[:, :H]
        out_ref[1] = z[:, H:]
    else:
        out_ref[...] = z


def _tc_dense(agg_split, norms, w, b2d, pre_col, post_col, split_out):
    grid = (N_PAD // BN,)
    if split_out:
        out_spec = pl.BlockSpec((NC, BN, H), lambda i: (0, i, 0))
        out_shape = jax.ShapeDtypeStruct((NC, N_PAD, H), jnp.float32)
    else:
        out_spec = pl.BlockSpec((BN, D), lambda i: (i, 0))
        out_shape = jax.ShapeDtypeStruct((N_PAD, D), jnp.float32)
    body = functools.partial(
        _dense_body, pre_col=pre_col, post_col=post_col, split_out=split_out
    )
    return pl.pallas_call(
        body,
        grid=grid,
        in_specs=[
            pl.BlockSpec((NC, BN, H), lambda i: (0, i, 0)),
            pl.BlockSpec((BN, 4), lambda i: (i, 0)),
            pl.BlockSpec((D, D), lambda i: (0, 0)),
            pl.BlockSpec((1, D), lambda i: (0, 0)),
        ],
        out_specs=out_spec,
        out_shape=out_shape,
    )(agg_split, norms, w, b2d)


def kernel(x, edge_index1, edge_index2, W1, b1, W2, b2):
    x = x.astype(jnp.float32)
    ei1 = edge_index1.astype(jnp.int32)
    ei2 = edge_index2.astype(jnp.int32)
    src1, dst1 = ei1[0], ei1[1]
    src2, dst2 = ei2[0], ei2[1]

    degp = _sc_degrees(src1, dst1, src2, dst2)
    degp_t = degp.transpose(2, 0, 1).reshape(N_PAD, 4 * NC * NS)
    sel = (jnp.arange(4 * NC * NS, dtype=jnp.int32)[:, None] // (NC * NS)
           == jnp.arange(4, dtype=jnp.int32)[None, :]).astype(jnp.float32)

    x_pad = jnp.pad(x, ((0, N_PAD - N_NODES), (0, 0)))
    h0_split, norms = _tc_norm_scale(degp_t, x_pad, sel)

    pad = jnp.full((E_PAD - N_EDGES,), N_NODES, jnp.int32)
    def packed(src, dst):
        srcp = jnp.concatenate([src, pad])
        dstp = jnp.concatenate([dst, pad])
        return ((dstp << 16) | srcp).reshape(NS * N_ECHUNK, EC)

    comb1 = packed(src1, dst1)
    comb2 = packed(src2, dst2)
    zrows = jnp.zeros((ROWS_PER_TILE, H), jnp.float32)

    agg1 = _sc_agg(h0_split, comb1, zrows)
    h1_split = _tc_dense(agg1, norms, W1, b1.reshape(1, D), 1, 2, True)
    agg2 = _sc_agg(h1_split, comb2, zrows)
    out = _tc_dense(agg2, norms, W2, b2.reshape(1, D), 3, None, False)
    return out[:N_NODES]


# speedup vs baseline: 1.6050x; 1.0014x over previous
import functools

import jax
import jax.numpy as jnp
from jax import lax
from jax.experimental import pallas as pl
from jax.experimental.pallas import tpu as pltpu
from jax.experimental.pallas import tpu_sc as plsc

N_NODES = 10000
N_PAD = 10240
N_EDGES = 320000
D = 128
H = 64
NC = 2
NS = 16

ROWS_PER_TILE = N_PAD // NS
EC = 128
N_ECHUNK = 160
E_PAD = NS * N_ECHUNK * EC

EDGES_PER_WORKER = N_EDGES // (NC * NS)
DEG_CHUNK = 2000
N_DEG_CHUNK = EDGES_PER_WORKER // DEG_CHUNK

BN = 1024


def _mesh():
    return plsc.VectorSubcoreMesh(
        core_axis_name="c", subcore_axis_name="s", num_cores=NC, num_subcores=NS
    )


_UNROLL = 5


def _deg_body(src1, dst1, src2, dst2, out, cnt, idx0, idx1, isem0, isem1):
    c = lax.axis_index("c")
    s = lax.axis_index("s")
    wid = s * NC + c
    zeros16 = jnp.zeros((16,), jnp.float32)
    ones16 = jnp.ones((16,), jnp.float32)

    def zbody(i, carry):
        for u in range(8):
            cnt[pl.ds((i * 8 + u) * 16, 16)] = zeros16
        return carry

    lax.fori_loop(0, (4 * N_PAD) // (16 * 8), zbody, None)

    refs = (src1, dst1, src2, dst2)
    bufs = (idx0, idx1)
    isem = (isem0, isem1)
    ebase = wid * EDGES_PER_WORKER
    seq = [(a, k) for a in range(4) for k in range(N_DEG_CHUNK)]

    def issue_idx(t, jb):
        a, k = seq[t]
        pltpu.async_copy(
            refs[a].at[pl.ds(ebase + k * DEG_CHUNK, DEG_CHUNK)], bufs[jb], isem[jb]
        )

    def wait_idx(jb):
        pltpu.make_async_copy(src1.at[pl.ds(0, DEG_CHUNK)], bufs[jb], isem[jb]).wait()

    issue_idx(0, 0)
    for t in range(len(seq)):
        jb = t % 2
        if t + 1 < len(seq):
            issue_idx(t + 1, (t + 1) % 2)
        wait_idx(jb)
        off = seq[t][0] * N_PAD

        def ibody(j, inner, jb=jb, off=off):
            for u in range(_UNROLL):
                idx = bufs[jb][pl.ds((j * _UNROLL + u) * 16, 16)]
                plsc.addupdate_scatter(cnt, [idx + off], ones16)
            return inner

        lax.fori_loop(0, DEG_CHUNK // (16 * _UNROLL), ibody, None)

    for a in range(4):
        pltpu.sync_copy(cnt.at[pl.ds(a * N_PAD, N_PAD)], out.at[a, wid])


def _sc_degrees(src1, dst1, src2, dst2):
    return pl.kernel(
        _deg_body,
        out_type=jax.ShapeDtypeStruct((4, NC * NS, N_PAD), jnp.float32),
        mesh=_mesh(),
        compiler_params=pltpu.CompilerParams(needs_layout_passes=False, use_tc_tiling_on_sc=False),
        scratch_types=[
            pltpu.VMEM((4 * N_PAD,), jnp.float32),
            pltpu.VMEM((DEG_CHUNK,), jnp.int32),
            pltpu.VMEM((DEG_CHUNK,), jnp.int32),
            pltpu.SemaphoreType.DMA,
            pltpu.SemaphoreType.DMA,
        ],
    )(src1, dst1, src2, dst2)


SB = 2
N_SCHUNK = N_ECHUNK // (2 * SB)
PHASE_CHUNKS = N_ECHUNK // 2


def _agg_body(h2, combr, zrows, out, sh_h, sh_agg, comb, src_i, dst_i,
              rows0, rows1, gsem0, gsem1, ssem0, ssem1):
    c = lax.axis_index("c")
    s = lax.axis_index("s")
    r0 = s * ROWS_PER_TILE
    pltpu.sync_copy(h2.at[c, pl.ds(r0, ROWS_PER_TILE)], sh_h.at[pl.ds(r0, ROWS_PER_TILE)])
    pltpu.sync_copy(zrows, sh_agg.at[pl.ds(r0, ROWS_PER_TILE)])
    plsc.subcore_barrier()

    rows = (rows0, rows1)
    gsem = (gsem0, gsem1)
    ssem = (ssem0, ssem1)
    mask16 = jnp.full((16,), 0xFFFF, jnp.int32)
    sh16 = jnp.full((16,), 16, jnp.int32)

    def unpack(sk, jb):
        for u in range(SB):
            for v in range(EC // 16):
                cv = comb[sk * SB + u, pl.ds(v * 16, 16)]
                src_i[jb, u, pl.ds(v * 16, 16)] = cv & mask16
                dst_i[jb, u, pl.ds(v * 16, 16)] = lax.shift_right_logical(cv, sh16)

    def issue_gather(sk, jb):
        unpack(sk, jb)
        for u in range(SB):
            pltpu.async_copy(
                sh_h.at[src_i.at[jb, u]], rows[jb].at[pl.ds(u * EC, EC)], gsem[jb]
            )

    def wait_gather(jb):
        pltpu.make_async_copy(h2.at[c, pl.ds(0, SB * EC)], rows[jb], gsem[jb]).wait()

    def issue_scatter(jb):
        for u in range(SB):
            pltpu.async_copy(
                rows[jb].at[pl.ds(u * EC, EC)], sh_agg.at[dst_i.at[jb, u]],
                ssem[jb], add=True,
            )

    def wait_scatter(jb):
        pltpu.make_async_copy(h2.at[c, pl.ds(0, SB * EC)], rows[jb], ssem[jb]).wait()

    for ph in range(2):
        pltpu.sync_copy(
            combr.at[pl.ds(s * N_ECHUNK + ph * PHASE_CHUNKS, PHASE_CHUNKS)], comb
        )
        issue_gather(0, 0)

        def gbody(g, carry):
            for j in (0, 1):
                sk = g * 2 + j
                nb = (j + 1) % 2

                @pl.when(jnp.logical_or(g > 0, j > 0))
                def _():
                    wait_scatter(nb)

                @pl.when(g * 2 + j + 1 < N_SCHUNK)
                def _():
                    issue_gather(sk + 1, nb)

                wait_gather(j)
                issue_scatter(j)
            return carry

        lax.fori_loop(0, N_SCHUNK // 2, gbody, None)
        wait_scatter(1)
    plsc.subcore_barrier()
    pltpu.sync_copy(sh_agg.at[pl.ds(r0, ROWS_PER_TILE)], out.at[c, pl.ds(r0, ROWS_PER_TILE)])


def _sc_agg(h_split, combr, zrows):
    return pl.kernel(
        _agg_body,
        out_type=jax.ShapeDtypeStruct((NC, N_PAD, H), jnp.float32),
        mesh=_mesh(),
        compiler_params=pltpu.CompilerParams(needs_layout_passes=False, use_tc_tiling_on_sc=False),
        scratch_types=[
            pltpu.VMEM_SHARED((N_PAD, H), jnp.float32),
            pltpu.VMEM_SHARED((N_PAD, H), jnp.float32),
            pltpu.VMEM((PHASE_CHUNKS, EC), jnp.int32),
            pltpu.VMEM((2, SB, EC), jnp.int32),
            pltpu.VMEM((2, SB, EC), jnp.int32),
            pltpu.VMEM((SB * EC, H), jnp.float32),
            pltpu.VMEM((SB * EC, H), jnp.float32),
            pltpu.SemaphoreType.DMA,
            pltpu.SemaphoreType.DMA,
            pltpu.SemaphoreType.DMA,
            pltpu.SemaphoreType.DMA,
        ],
    )(h_split, combr, zrows)


def _norm_scale_body(degp_ref, x_ref, sel_ref, h0_ref, norms_ref):
    deg = jnp.dot(
        degp_ref[...], sel_ref[...],
        preferred_element_type=jnp.float32,
        precision=lax.Precision.HIGHEST,
    )
    norms = jnp.where(deg > 0, lax.rsqrt(jnp.maximum(deg, 1.0)), 0.0)
    norms_ref[...] = norms
    h0 = x_ref[...] * norms[:, 0:1]
    h0_ref[0] = h0[:, :H]
    h0_ref[1] = h0[:, H:]


def _tc_norm_scale(degp_t, x, sel):
    grid = (N_PAD // BN,)
    return pl.pallas_call(
        _norm_scale_body,
        grid=grid,
        in_specs=[
            pl.BlockSpec((BN, D), lambda i: (i, 0)),
            pl.BlockSpec((BN, D), lambda i: (i, 0)),
            pl.BlockSpec((D, 4), lambda i: (0, 0)),
        ],
        out_specs=[
            pl.BlockSpec((NC, BN, H), lambda i: (0, i, 0)),
            pl.BlockSpec((BN, 4), lambda i: (i, 0)),
        ],
        out_shape=[
            jax.ShapeDtypeStruct((NC, N_PAD, H), jnp.float32),
            jax.ShapeDtypeStruct((N_PAD, 4), jnp.float32),
        ],
    )(degp_t, x, sel)


def _dense_body(agg_ref, norms_ref, w_ref, b_ref, out_ref, *, pre_col, post_col,
                split_out):
    z = jnp.concatenate([agg_ref[0], agg_ref[1]], axis=1)
    z = z * norms_ref[:, pre_col:pre_col + 1]
    z = jnp.dot(
        z, w_ref[...],
        preferred_element_type=jnp.float32,
        precision=lax.Precision.HIGHEST,
    ) + b_ref[...]
    if post_col is not None:
        z = z * norms_ref[:, post_col:post_col + 1]
    z = jnp.maximum(z, 0.0)
    if split_out:
        out_ref[0] = z[:, :H]
        out_ref[1] = z[:, H:]
    else:
        out_ref[...] = z


def _tc_dense(agg_split, norms, w, b2d, pre_col, post_col, split_out):
    grid = (N_PAD // BN,)
    if split_out:
        out_spec = pl.BlockSpec((NC, BN, H), lambda i: (0, i, 0))
        out_shape = jax.ShapeDtypeStruct((NC, N_PAD, H), jnp.float32)
    else:
        out_spec = pl.BlockSpec((BN, D), lambda i: (i, 0))
        out_shape = jax.ShapeDtypeStruct((N_PAD, D), jnp.float32)
    body = functools.partial(
        _dense_body, pre_col=pre_col, post_col=post_col, split_out=split_out
    )
    return pl.pallas_call(
        body,
        grid=grid,
        in_specs=[
            pl.BlockSpec((NC, BN, H), lambda i: (0, i, 0)),
            pl.BlockSpec((BN, 4), lambda i: (i, 0)),
            pl.BlockSpec((D, D), lambda i: (0, 0)),
            pl.BlockSpec((1, D), lambda i: (0, 0)),
        ],
        out_specs=out_spec,
        out_shape=out_shape,
    )(agg_split, norms, w, b2d)


def kernel(x, edge_index1, edge_index2, W1, b1, W2, b2):
    x = x.astype(jnp.float32)
    ei1 = edge_index1.astype(jnp.int32)
    ei2 = edge_index2.astype(jnp.int32)
    src1, dst1 = ei1[0], ei1[1]
    src2, dst2 = ei2[0], ei2[1]

    degp = _sc_degrees(src1, dst1, src2, dst2)
    degp_t = degp.transpose(2, 0, 1).reshape(N_PAD, 4 * NC * NS)
    sel = (jnp.arange(4 * NC * NS, dtype=jnp.int32)[:, None] // (NC * NS)
           == jnp.arange(4, dtype=jnp.int32)[None, :]).astype(jnp.float32)

    x_pad = jnp.pad(x, ((0, N_PAD - N_NODES), (0, 0)))
    h0_split, norms = _tc_norm_scale(degp_t, x_pad, sel)

    pad = jnp.full((E_PAD - N_EDGES,), N_NODES, jnp.int32)
    def packed(src, dst):
        srcp = jnp.concatenate([src, pad])
        dstp = jnp.concatenate([dst, pad])
        return ((dstp << 16) | srcp).reshape(NS * N_ECHUNK, EC)

    comb1 = packed(src1, dst1)
    comb2 = packed(src2, dst2)
    zrows = jnp.zeros((ROWS_PER_TILE, H), jnp.float32)

    agg1 = _sc_agg(h0_split, comb1, zrows)
    h1_split = _tc_dense(agg1, norms, W1, b1.reshape(1, D), 1, 2, True)
    agg2 = _sc_agg(h1_split, comb2, zrows)
    out = _tc_dense(agg2, norms, W2, b2.reshape(1, D), 3, None, False)
    return out[:N_NODES]


# speedup vs baseline: 2.2094x; 1.3766x over previous
import functools

import jax
import jax.numpy as jnp
from jax import lax
from jax.experimental import pallas as pl
from jax.experimental.pallas import tpu as pltpu
from jax.experimental.pallas import tpu_sc as plsc

N_NODES = 10000
N_PAD = 10240
N_EDGES = 320000
D = 128
H = 64
NC = 2
NS = 16

ROWS_PER_TILE = N_PAD // NS
EC = 128
N_ECHUNK = 160
E_PAD = NS * N_ECHUNK * EC

EDGES_PER_WORKER = N_EDGES // (NC * NS)
DEG_CHUNK = 2000
N_DEG_CHUNK = EDGES_PER_WORKER // DEG_CHUNK

BN = 1024


def _mesh():
    return plsc.VectorSubcoreMesh(
        core_axis_name="c", subcore_axis_name="s", num_cores=NC, num_subcores=NS
    )


_UNROLL = 5


def _deg_body(src1, dst1, src2, dst2, out, cnt, idx0, idx1, isem0, isem1):
    c = lax.axis_index("c")
    s = lax.axis_index("s")
    wid = s * NC + c
    zeros16 = jnp.zeros((16,), jnp.float32)
    ones16 = jnp.ones((16,), jnp.float32)

    def zbody(i, carry):
        for u in range(8):
            cnt[pl.ds((i * 8 + u) * 16, 16)] = zeros16
        return carry

    lax.fori_loop(0, (4 * N_PAD) // (16 * 8), zbody, None)

    refs = (src1, dst1, src2, dst2)
    bufs = (idx0, idx1)
    isem = (isem0, isem1)
    ebase = wid * EDGES_PER_WORKER
    seq = [(a, k) for a in range(4) for k in range(N_DEG_CHUNK)]

    def issue_idx(t, jb):
        a, k = seq[t]
        pltpu.async_copy(
            refs[a].at[pl.ds(ebase + k * DEG_CHUNK, DEG_CHUNK)], bufs[jb], isem[jb]
        )

    def wait_idx(jb):
        pltpu.make_async_copy(src1.at[pl.ds(0, DEG_CHUNK)], bufs[jb], isem[jb]).wait()

    issue_idx(0, 0)
    for t in range(len(seq)):
        jb = t % 2
        if t + 1 < len(seq):
            issue_idx(t + 1, (t + 1) % 2)
        wait_idx(jb)
        off = seq[t][0] * N_PAD

        def ibody(j, inner, jb=jb, off=off):
            for u in range(_UNROLL):
                idx = bufs[jb][pl.ds((j * _UNROLL + u) * 16, 16)]
                plsc.addupdate_scatter(cnt, [idx + off], ones16)
            return inner

        lax.fori_loop(0, DEG_CHUNK // (16 * _UNROLL), ibody, None)

    for a in range(4):
        pltpu.sync_copy(cnt.at[pl.ds(a * N_PAD, N_PAD)], out.at[a, wid])


def _sc_degrees(src1, dst1, src2, dst2):
    return pl.kernel(
        _deg_body,
        out_type=jax.ShapeDtypeStruct((4, NC * NS, N_PAD), jnp.float32),
        mesh=_mesh(),
        compiler_params=pltpu.CompilerParams(needs_layout_passes=False, use_tc_tiling_on_sc=False),
        scratch_types=[
            pltpu.VMEM((4 * N_PAD,), jnp.float32),
            pltpu.VMEM((DEG_CHUNK,), jnp.int32),
            pltpu.VMEM((DEG_CHUNK,), jnp.int32),
            pltpu.SemaphoreType.DMA,
            pltpu.SemaphoreType.DMA,
        ],
    )(src1, dst1, src2, dst2)


SB = 2
N_SCHUNK = N_ECHUNK // (2 * SB)
PHASE_CHUNKS = N_ECHUNK // 2


def _agg_body(h2, combr, zrows, out, sh_h, sh_agg, comb, src_i, dst_i,
              rows0, rows1, gsem0, gsem1, ssem0, ssem1):
    c = lax.axis_index("c")
    s = lax.axis_index("s")
    r0 = s * ROWS_PER_TILE
    pltpu.sync_copy(h2.at[c, pl.ds(r0, ROWS_PER_TILE)], sh_h.at[pl.ds(r0, ROWS_PER_TILE)])
    pltpu.sync_copy(zrows, sh_agg.at[pl.ds(r0, ROWS_PER_TILE)])
    plsc.subcore_barrier()

    rows = (rows0, rows1)
    gsem = (gsem0, gsem1)
    ssem = (ssem0, ssem1)
    mask16 = jnp.full((16,), 0xFFFF, jnp.int32)
    sh16 = jnp.full((16,), 16, jnp.int32)

    def unpack(sk, jb):
        for u in range(SB):
            for v in range(EC // 16):
                cv = comb[sk * SB + u, pl.ds(v * 16, 16)]
                src_i[jb, u, pl.ds(v * 16, 16)] = cv & mask16
                dst_i[jb, u, pl.ds(v * 16, 16)] = lax.shift_right_logical(cv, sh16)

    def issue_gather(sk, jb):
        unpack(sk, jb)
        for u in range(SB):
            pltpu.async_copy(
                sh_h.at[src_i.at[jb, u]], rows[jb].at[pl.ds(u * EC, EC)], gsem[jb]
            )

    def wait_gather(jb):
        pltpu.make_async_copy(h2.at[c, pl.ds(0, SB * EC)], rows[jb], gsem[jb]).wait()

    def issue_scatter(jb):
        for u in range(SB):
            pltpu.async_copy(
                rows[jb].at[pl.ds(u * EC, EC)], sh_agg.at[dst_i.at[jb, u]],
                ssem[jb], add=True,
            )

    def wait_scatter(jb):
        pltpu.make_async_copy(h2.at[c, pl.ds(0, SB * EC)], rows[jb], ssem[jb]).wait()

    for ph in range(2):
        pltpu.sync_copy(
            combr.at[pl.ds(s * N_ECHUNK + ph * PHASE_CHUNKS, PHASE_CHUNKS)], comb
        )
        issue_gather(0, 0)

        def gbody(g, carry):
            for j in (0, 1):
                sk = g * 2 + j
                nb = (j + 1) % 2

                @pl.when(jnp.logical_or(g > 0, j > 0))
                def _():
                    wait_scatter(nb)

                @pl.when(g * 2 + j + 1 < N_SCHUNK)
                def _():
                    issue_gather(sk + 1, nb)

                wait_gather(j)
                issue_scatter(j)
            return carry

        lax.fori_loop(0, N_SCHUNK // 2, gbody, None)
        wait_scatter(1)
    plsc.subcore_barrier()
    pltpu.sync_copy(sh_agg.at[pl.ds(r0, ROWS_PER_TILE)], out.at[c, pl.ds(r0, ROWS_PER_TILE)])


def _sc_agg(h_split, combr, zrows):
    return pl.kernel(
        _agg_body,
        out_type=jax.ShapeDtypeStruct((NC, N_PAD, H), jnp.bfloat16),
        mesh=_mesh(),
        compiler_params=pltpu.CompilerParams(needs_layout_passes=False, use_tc_tiling_on_sc=False),
        scratch_types=[
            pltpu.VMEM_SHARED((N_PAD, H), jnp.bfloat16),
            pltpu.VMEM_SHARED((N_PAD, H), jnp.bfloat16),
            pltpu.VMEM((PHASE_CHUNKS, EC), jnp.int32),
            pltpu.VMEM((2, SB, EC), jnp.int32),
            pltpu.VMEM((2, SB, EC), jnp.int32),
            pltpu.VMEM((SB * EC, H), jnp.bfloat16),
            pltpu.VMEM((SB * EC, H), jnp.bfloat16),
            pltpu.SemaphoreType.DMA,
            pltpu.SemaphoreType.DMA,
            pltpu.SemaphoreType.DMA,
            pltpu.SemaphoreType.DMA,
        ],
    )(h_split, combr, zrows)


def _norm_scale_body(degp_ref, x_ref, sel_ref, h0_ref, norms_ref):
    deg = jnp.dot(
        degp_ref[...], sel_ref[...],
        preferred_element_type=jnp.float32,
        precision=lax.Precision.HIGHEST,
    )
    norms = jnp.where(deg > 0, lax.rsqrt(jnp.maximum(deg, 1.0)), 0.0)
    norms_ref[...] = norms
    h0 = (x_ref[...] * norms[:, 0:1]).astype(jnp.bfloat16)
    h0_ref[0] = h0[:, :H]
    h0_ref[1] = h0[:, H:]


def _tc_norm_scale(degp_t, x, sel):
    grid = (N_PAD // BN,)
    return pl.pallas_call(
        _norm_scale_body,
        grid=grid,
        in_specs=[
            pl.BlockSpec((BN, D), lambda i: (i, 0)),
            pl.BlockSpec((BN, D), lambda i: (i, 0)),
            pl.BlockSpec((D, 4), lambda i: (0, 0)),
        ],
        out_specs=[
            pl.BlockSpec((NC, BN, H), lambda i: (0, i, 0)),
            pl.BlockSpec((BN, 4), lambda i: (i, 0)),
        ],
        out_shape=[
            jax.ShapeDtypeStruct((NC, N_PAD, H), jnp.bfloat16),
            jax.ShapeDtypeStruct((N_PAD, 4), jnp.float32),
        ],
    )(degp_t, x, sel)


def _dense_body(agg_ref, norms_ref, w_ref, b_ref, out_ref, *, pre_col, post_col,
                split_out):
    z = jnp.concatenate([agg_ref[0], agg_ref[1]], axis=1).astype(jnp.float32)
    z = z * norms_ref[:, pre_col:pre_col + 1]
    z = jnp.dot(
        z, w_ref[...],
        preferred_element_type=jnp.float32,
        precision=lax.Precision.HIGHEST,
    ) + b_ref[...]
    if post_col is not None:
        z = z * norms_ref[:, post_col:post_col + 1]
    z = jnp.maximum(z, 0.0)
    if split_out:
        zb = z.astype(jnp.bfloat16)
        out_ref[0] = zb[:, :H]
        out_ref[1] = zb[:, H:]
    else:
        out_ref[...] = z


def _tc_dense(agg_split, norms, w, b2d, pre_col, post_col, split_out):
    grid = (N_PAD // BN,)
    if split_out:
        out_spec = pl.BlockSpec((NC, BN, H), lambda i: (0, i, 0))
        out_shape = jax.ShapeDtypeStruct((NC, N_PAD, H), jnp.bfloat16)
    else:
        out_spec = pl.BlockSpec((BN, D), lambda i: (i, 0))
        out_shape = jax.ShapeDtypeStruct((N_PAD, D), jnp.float32)
    body = functools.partial(
        _dense_body, pre_col=pre_col, post_col=post_col, split_out=split_out
    )
    return pl.pallas_call(
        body,
        grid=grid,
        in_specs=[
            pl.BlockSpec((NC, BN, H), lambda i: (0, i, 0)),
            pl.BlockSpec((BN, 4), lambda i: (i, 0)),
            pl.BlockSpec((D, D), lambda i: (0, 0)),
            pl.BlockSpec((1, D), lambda i: (0, 0)),
        ],
        out_specs=out_spec,
        out_shape=out_shape,
    )(agg_split, norms, w, b2d)


def kernel(x, edge_index1, edge_index2, W1, b1, W2, b2):
    x = x.astype(jnp.float32)
    ei1 = edge_index1.astype(jnp.int32)
    ei2 = edge_index2.astype(jnp.int32)
    src1, dst1 = ei1[0], ei1[1]
    src2, dst2 = ei2[0], ei2[1]

    degp = _sc_degrees(src1, dst1, src2, dst2)
    degp_t = degp.transpose(2, 0, 1).reshape(N_PAD, 4 * NC * NS)
    sel = (jnp.arange(4 * NC * NS, dtype=jnp.int32)[:, None] // (NC * NS)
           == jnp.arange(4, dtype=jnp.int32)[None, :]).astype(jnp.float32)

    x_pad = jnp.pad(x, ((0, N_PAD - N_NODES), (0, 0)))
    h0_split, norms = _tc_norm_scale(degp_t, x_pad, sel)

    pad = jnp.full((E_PAD - N_EDGES,), N_NODES, jnp.int32)
    def packed(src, dst):
        srcp = jnp.concatenate([src, pad])
        dstp = jnp.concatenate([dst, pad])
        return ((dstp << 16) | srcp).reshape(NS * N_ECHUNK, EC)

    comb1 = packed(src1, dst1)
    comb2 = packed(src2, dst2)
    zrows = jnp.zeros((ROWS_PER_TILE, H), jnp.bfloat16)

    agg1 = _sc_agg(h0_split, comb1, zrows)
    h1_split = _tc_dense(agg1, norms, W1, b1.reshape(1, D), 1, 2, True)
    agg2 = _sc_agg(h1_split, comb2, zrows)
    out = _tc_dense(agg2, norms, W2, b2.reshape(1, D), 3, None, False)
    return out[:N_NODES]


# speedup vs baseline: 2.2988x; 1.0405x over previous
import functools

import jax
import jax.numpy as jnp
from jax import lax
from jax.experimental import pallas as pl
from jax.experimental.pallas import tpu as pltpu
from jax.experimental.pallas import tpu_sc as plsc

N_NODES = 10000
N_PAD = 10240
N_EDGES = 320000
D = 128
H = 64
NC = 2
NS = 16

ROWS_PER_TILE = N_PAD // NS
EC = 128
N_ECHUNK = 160
E_PAD = NS * N_ECHUNK * EC

EDGES_PER_WORKER = N_EDGES // (NC * NS)
DEG_CHUNK = 2000
N_DEG_CHUNK = EDGES_PER_WORKER // DEG_CHUNK

BN = 2000


def _mesh():
    return plsc.VectorSubcoreMesh(
        core_axis_name="c", subcore_axis_name="s", num_cores=NC, num_subcores=NS
    )


_UNROLL = 5


def _deg_body(src1, dst1, src2, dst2, out, cnt, idx0, idx1, isem0, isem1):
    c = lax.axis_index("c")
    s = lax.axis_index("s")
    wid = s * NC + c
    zeros16 = jnp.zeros((16,), jnp.float32)
    ones16 = jnp.ones((16,), jnp.float32)

    def zbody(i, carry):
        for u in range(8):
            cnt[pl.ds((i * 8 + u) * 16, 16)] = zeros16
        return carry

    lax.fori_loop(0, (4 * N_PAD) // (16 * 8), zbody, None)

    refs = (src1, dst1, src2, dst2)
    bufs = (idx0, idx1)
    isem = (isem0, isem1)
    ebase = wid * EDGES_PER_WORKER
    seq = [(a, k) for a in range(4) for k in range(N_DEG_CHUNK)]

    def issue_idx(t, jb):
        a, k = seq[t]
        pltpu.async_copy(
            refs[a].at[pl.ds(ebase + k * DEG_CHUNK, DEG_CHUNK)], bufs[jb], isem[jb]
        )

    def wait_idx(jb):
        pltpu.make_async_copy(src1.at[pl.ds(0, DEG_CHUNK)], bufs[jb], isem[jb]).wait()

    issue_idx(0, 0)
    for t in range(len(seq)):
        jb = t % 2
        if t + 1 < len(seq):
            issue_idx(t + 1, (t + 1) % 2)
        wait_idx(jb)
        off = seq[t][0] * N_PAD

        def ibody(j, inner, jb=jb, off=off):
            for u in range(_UNROLL):
                idx = bufs[jb][pl.ds((j * _UNROLL + u) * 16, 16)]
                plsc.addupdate_scatter(cnt, [idx + off], ones16)
            return inner

        lax.fori_loop(0, DEG_CHUNK // (16 * _UNROLL), ibody, None)

    for a in range(4):
        pltpu.sync_copy(cnt.at[pl.ds(a * N_PAD, N_PAD)], out.at[a, wid])


def _sc_degrees(src1, dst1, src2, dst2):
    return pl.kernel(
        _deg_body,
        out_type=jax.ShapeDtypeStruct((4, NC * NS, N_PAD), jnp.float32),
        mesh=_mesh(),
        compiler_params=pltpu.CompilerParams(needs_layout_passes=False, use_tc_tiling_on_sc=False),
        scratch_types=[
            pltpu.VMEM((4 * N_PAD,), jnp.float32),
            pltpu.VMEM((DEG_CHUNK,), jnp.int32),
            pltpu.VMEM((DEG_CHUNK,), jnp.int32),
            pltpu.SemaphoreType.DMA,
            pltpu.SemaphoreType.DMA,
        ],
    )(src1, dst1, src2, dst2)


SB = 2
N_SCHUNK = N_ECHUNK // (2 * SB)
PHASE_CHUNKS = N_ECHUNK // 2


def _agg_body(h2, combr, zrows, out, sh_h, sh_agg, comb, src_i, dst_i,
              rows0, rows1, gsem0, gsem1, ssem0, ssem1):
    c = lax.axis_index("c")
    s = lax.axis_index("s")
    r0 = s * ROWS_PER_TILE
    pltpu.sync_copy(h2.at[c, pl.ds(r0, ROWS_PER_TILE)], sh_h.at[pl.ds(r0, ROWS_PER_TILE)])
    pltpu.sync_copy(zrows, sh_agg.at[pl.ds(r0, ROWS_PER_TILE)])
    plsc.subcore_barrier()

    rows = (rows0, rows1)
    gsem = (gsem0, gsem1)
    ssem = (ssem0, ssem1)
    mask16 = jnp.full((16,), 0xFFFF, jnp.int32)
    sh16 = jnp.full((16,), 16, jnp.int32)

    def unpack(sk, jb):
        for u in range(SB):
            for v in range(EC // 16):
                cv = comb[sk * SB + u, pl.ds(v * 16, 16)]
                src_i[jb, u, pl.ds(v * 16, 16)] = cv & mask16
                dst_i[jb, u, pl.ds(v * 16, 16)] = lax.shift_right_logical(cv, sh16)

    def issue_gather(sk, jb):
        unpack(sk, jb)
        for u in range(SB):
            pltpu.async_copy(
                sh_h.at[src_i.at[jb, u]], rows[jb].at[pl.ds(u * EC, EC)], gsem[jb]
            )

    def wait_gather(jb):
        pltpu.make_async_copy(h2.at[c, pl.ds(0, SB * EC)], rows[jb], gsem[jb]).wait()

    def issue_scatter(jb):
        for u in range(SB):
            pltpu.async_copy(
                rows[jb].at[pl.ds(u * EC, EC)], sh_agg.at[dst_i.at[jb, u]],
                ssem[jb], add=True,
            )

    def wait_scatter(jb):
        pltpu.make_async_copy(h2.at[c, pl.ds(0, SB * EC)], rows[jb], ssem[jb]).wait()

    for ph in range(2):
        pltpu.sync_copy(
            combr.at[pl.ds(s * N_ECHUNK + ph * PHASE_CHUNKS, PHASE_CHUNKS)], comb
        )
        issue_gather(0, 0)

        def gbody(g, carry):
            for j in (0, 1):
                sk = g * 2 + j
                nb = (j + 1) % 2

                @pl.when(jnp.logical_or(g > 0, j > 0))
                def _():
                    wait_scatter(nb)

                @pl.when(g * 2 + j + 1 < N_SCHUNK)
                def _():
                    issue_gather(sk + 1, nb)

                wait_gather(j)
                issue_scatter(j)
            return carry

        lax.fori_loop(0, N_SCHUNK // 2, gbody, None)
        wait_scatter(1)
    plsc.subcore_barrier()
    pltpu.sync_copy(sh_agg.at[pl.ds(r0, ROWS_PER_TILE)], out.at[c, pl.ds(r0, ROWS_PER_TILE)])


def _sc_agg(h_split, combr, zrows):
    return pl.kernel(
        _agg_body,
        out_type=jax.ShapeDtypeStruct((NC, N_PAD, H), jnp.bfloat16),
        mesh=_mesh(),
        compiler_params=pltpu.CompilerParams(needs_layout_passes=False, use_tc_tiling_on_sc=False),
        scratch_types=[
            pltpu.VMEM_SHARED((N_PAD, H), jnp.bfloat16),
            pltpu.VMEM_SHARED((N_PAD, H), jnp.bfloat16),
            pltpu.VMEM((PHASE_CHUNKS, EC), jnp.int32),
            pltpu.VMEM((2, SB, EC), jnp.int32),
            pltpu.VMEM((2, SB, EC), jnp.int32),
            pltpu.VMEM((SB * EC, H), jnp.bfloat16),
            pltpu.VMEM((SB * EC, H), jnp.bfloat16),
            pltpu.SemaphoreType.DMA,
            pltpu.SemaphoreType.DMA,
            pltpu.SemaphoreType.DMA,
            pltpu.SemaphoreType.DMA,
        ],
    )(h_split, combr, zrows)


def _norm_scale_body(degp_ref, x_ref, sel_ref, h0_ref, norms_ref):
    deg = jnp.dot(
        degp_ref[...], sel_ref[...],
        preferred_element_type=jnp.float32,
        precision=lax.Precision.HIGHEST,
    )
    norms = jnp.where(deg > 0, lax.rsqrt(jnp.maximum(deg, 1.0)), 0.0)
    norms_ref[...] = norms
    h0 = (x_ref[...] * norms[:, 0:1]).astype(jnp.bfloat16)
    h0_ref[0] = h0[:, :H]
    h0_ref[1] = h0[:, H:]


def _tc_norm_scale(degp_t, x, sel):
    grid = (N_NODES // BN,)
    return pl.pallas_call(
        _norm_scale_body,
        grid=grid,
        in_specs=[
            pl.BlockSpec((BN, D), lambda i: (i, 0)),
            pl.BlockSpec((BN, D), lambda i: (i, 0)),
            pl.BlockSpec((D, 4), lambda i: (0, 0)),
        ],
        out_specs=[
            pl.BlockSpec((NC, BN, H), lambda i: (0, i, 0)),
            pl.BlockSpec((BN, 4), lambda i: (i, 0)),
        ],
        out_shape=[
            jax.ShapeDtypeStruct((NC, N_PAD, H), jnp.bfloat16),
            jax.ShapeDtypeStruct((N_NODES, 4), jnp.float32),
        ],
    )(degp_t, x, sel)


def _dense_body(agg_ref, norms_ref, w_ref, b_ref, out_ref, *, pre_col, post_col,
                split_out):
    z = jnp.concatenate([agg_ref[0], agg_ref[1]], axis=1).astype(jnp.float32)
    z = z * norms_ref[:, pre_col:pre_col + 1]
    z = jnp.dot(
        z, w_ref[...],
        preferred_element_type=jnp.float32,
        precision=lax.Precision.HIGHEST,
    ) + b_ref[...]
    if post_col is not None:
        z = z * norms_ref[:, post_col:post_col + 1]
    z = jnp.maximum(z, 0.0)
    if split_out:
        zb = z.astype(jnp.bfloat16)
        out_ref[0] = zb[:, :H]
        out_ref[1] = zb[:, H:]
    else:
        out_ref[...] = z


def _tc_dense(agg_split, norms, w, b2d, pre_col, post_col, split_out):
    grid = (N_NODES // BN,)
    if split_out:
        out_spec = pl.BlockSpec((NC, BN, H), lambda i: (0, i, 0))
        out_shape = jax.ShapeDtypeStruct((NC, N_PAD, H), jnp.bfloat16)
    else:
        out_spec = pl.BlockSpec((BN, D), lambda i: (i, 0))
        out_shape = jax.ShapeDtypeStruct((N_NODES, D), jnp.float32)
    body = functools.partial(
        _dense_body, pre_col=pre_col, post_col=post_col, split_out=split_out
    )
    return pl.pallas_call(
        body,
        grid=grid,
        in_specs=[
            pl.BlockSpec((NC, BN, H), lambda i: (0, i, 0)),
            pl.BlockSpec((BN, 4), lambda i: (i, 0)),
            pl.BlockSpec((D, D), lambda i: (0, 0)),
            pl.BlockSpec((1, D), lambda i: (0, 0)),
        ],
        out_specs=out_spec,
        out_shape=out_shape,
    )(agg_split, norms, w, b2d)


def kernel(x, edge_index1, edge_index2, W1, b1, W2, b2):
    x = x.astype(jnp.float32)
    ei1 = edge_index1.astype(jnp.int32)
    ei2 = edge_index2.astype(jnp.int32)
    src1, dst1 = ei1[0], ei1[1]
    src2, dst2 = ei2[0], ei2[1]

    degp = _sc_degrees(src1, dst1, src2, dst2)
    degp_t = degp.transpose(2, 0, 1).reshape(N_PAD, 4 * NC * NS)
    sel = (jnp.arange(4 * NC * NS, dtype=jnp.int32)[:, None] // (NC * NS)
           == jnp.arange(4, dtype=jnp.int32)[None, :]).astype(jnp.float32)

    h0_split, norms = _tc_norm_scale(degp_t, x, sel)

    pad = jnp.full((E_PAD - N_EDGES,), N_NODES, jnp.int32)
    def packed(src, dst):
        srcp = jnp.concatenate([src, pad])
        dstp = jnp.concatenate([dst, pad])
        return ((dstp << 16) | srcp).reshape(NS * N_ECHUNK, EC)

    comb1 = packed(src1, dst1)
    comb2 = packed(src2, dst2)
    zrows = jnp.zeros((ROWS_PER_TILE, H), jnp.bfloat16)

    agg1 = _sc_agg(h0_split, comb1, zrows)
    h1_split = _tc_dense(agg1, norms, W1, b1.reshape(1, D), 1, 2, True)
    agg2 = _sc_agg(h1_split, comb2, zrows)
    return _tc_dense(agg2, norms, W2, b2.reshape(1, D), 3, None, False)
